# baseline (speedup 1.0000x reference)
import functools

import jax
import jax.numpy as jnp
from jax import lax
from jax.experimental import pallas as pl
from jax.experimental.pallas import tpu as pltpu
from jax.experimental.pallas import tpu_sc as plsc

D_MODEL = 1024
MAX_SEQ = 2048
BATCH = 16
N_ROWS = BATCH * MAX_SEQ
NUM_WORKERS = 32
ROWS_PER_W = N_ROWS // NUM_WORKERS
CHUNK = 64
NCHUNK = ROWS_PER_W // CHUNK

_mesh = plsc.VectorSubcoreMesh(core_axis_name="c", subcore_axis_name="s")


@functools.partial(
    pl.kernel,
    mesh=_mesh,
    out_type=jax.ShapeDtypeStruct((N_ROWS, D_MODEL), jnp.float32),
    scratch_types=[
        pltpu.VMEM((16,), jnp.int32),
        pltpu.VMEM((CHUNK,), jnp.int32),
        pltpu.VMEM((CHUNK, D_MODEL), jnp.float32),
        pltpu.SemaphoreType.DMA,
    ],
)
def _pe_lookup(len_hbm, pe_hbm, out_hbm, lens_v, idx_v, rows_v, sem):
    cid = lax.axis_index("c")
    sid = lax.axis_index("s")
    wid = sid * 2 + cid
    t_base = (wid % 2) * (MAX_SEQ // 2)
    row_base = wid * ROWS_PER_W

    pltpu.sync_copy(len_hbm.at[wid], lens_v)
    l_vec = lens_v[...]
    iota16 = lax.broadcasted_iota(jnp.int32, (16,), 0)

    def chunk_body(g, carry):
        t0 = t_base + g * CHUNK
        for j in range(CHUNK // 16):
            t = t0 + j * 16 + iota16
            idx_v[pl.ds(j * 16, 16)] = jnp.where(t < l_vec, t + 1, 0)
        pltpu.async_copy(pe_hbm.at[idx_v], rows_v, sem).wait()
        pltpu.sync_copy(rows_v, out_hbm.at[pl.ds(row_base + g * CHUNK, CHUNK)])
        return carry

    lax.fori_loop(0, NCHUNK, chunk_body, 0)


def kernel(input_len, position_encoding):
    lens_w = jnp.repeat(input_len.astype(jnp.int32), 2)
    lens_w = jnp.broadcast_to(lens_w[:, None], (NUM_WORKERS, 16))
    out = _pe_lookup(lens_w, position_encoding)
    return out.reshape(BATCH, MAX_SEQ, D_MODEL)
